# baseline (speedup 1.0000x reference)
import functools

import jax
import jax.numpy as jnp
from jax import lax
from jax.experimental import pallas as pl
from jax.experimental.pallas import tpu as pltpu
from jax.experimental.pallas import tpu_sc as plsc

COL_A = 17
COL_B = 503

N_ROWS = 16384
N_COLS = 1024

NUM_CORES = 2
NUM_SUBCORES = 16
NUM_WORKERS = NUM_CORES * NUM_SUBCORES
ROWS_PER_WORKER = N_ROWS // NUM_WORKERS
CHUNK = 64
NUM_CHUNKS = ROWS_PER_WORKER // CHUNK

_mesh = plsc.VectorSubcoreMesh(
    core_axis_name="c",
    subcore_axis_name="s",
    num_cores=NUM_CORES,
    num_subcores=NUM_SUBCORES,
)


@functools.partial(
    pl.kernel,
    out_type=jax.ShapeDtypeStruct((N_ROWS * N_COLS,), jnp.float32),
    mesh=_mesh,
    scratch_types=[pltpu.VMEM((CHUNK * N_COLS,), jnp.float32)],
    compiler_params=pltpu.CompilerParams(
        use_tc_tiling_on_sc=False, needs_layout_passes=False
    ),
)
def _swap_columns(x_hbm, out_hbm, buf):
    wid = lax.axis_index("s") * NUM_CORES + lax.axis_index("c")
    base = wid * (ROWS_PER_WORKER * N_COLS)

    def body(i, carry):
        off = base + i * (CHUNK * N_COLS)
        pltpu.sync_copy(x_hbm.at[pl.ds(off, CHUNK * N_COLS)], buf)
        for g in range(CHUNK // 16):
            row_off = (lax.iota(jnp.int32, 16) + (g * 16)) * N_COLS
            idx_a = row_off + COL_A
            idx_b = row_off + COL_B
            vals_a = plsc.load_gather(buf, [idx_a])
            vals_b = plsc.load_gather(buf, [idx_b])
            plsc.store_scatter(buf, [idx_a], vals_b)
            plsc.store_scatter(buf, [idx_b], vals_a)
        pltpu.sync_copy(buf, out_hbm.at[pl.ds(off, CHUNK * N_COLS)])
        return carry

    lax.fori_loop(0, NUM_CHUNKS, body, 0)


def kernel(X):
    return _swap_columns(X.reshape(-1)).reshape(N_ROWS, N_COLS)


# speedup vs baseline: 1.0167x; 1.0167x over previous
import functools

import jax
import jax.numpy as jnp
from jax import lax
from jax.experimental import pallas as pl
from jax.experimental.pallas import tpu as pltpu
from jax.experimental.pallas import tpu_sc as plsc

COL_A = 17
COL_B = 503

N_ROWS = 16384
N_COLS = 1024

NUM_CORES = 2
NUM_SUBCORES = 16
NUM_WORKERS = NUM_CORES * NUM_SUBCORES
ROWS_PER_WORKER = N_ROWS // NUM_WORKERS
CHUNK = 32
NUM_CHUNKS = ROWS_PER_WORKER // CHUNK
NBUF = 3
LAG = NBUF - 1

_mesh = plsc.VectorSubcoreMesh(
    core_axis_name="c",
    subcore_axis_name="s",
    num_cores=NUM_CORES,
    num_subcores=NUM_SUBCORES,
)


def _fix_columns(buf):
    for g in range(CHUNK // 16):
        row_off = (lax.iota(jnp.int32, 16) + (g * 16)) * N_COLS
        idx_a = row_off + COL_A
        idx_b = row_off + COL_B
        vals_a = plsc.load_gather(buf, [idx_a])
        vals_b = plsc.load_gather(buf, [idx_b])
        plsc.store_scatter(buf, [idx_a], vals_b)
        plsc.store_scatter(buf, [idx_b], vals_a)


@functools.partial(
    pl.kernel,
    out_type=jax.ShapeDtypeStruct((N_ROWS * N_COLS,), jnp.float32),
    mesh=_mesh,
    scratch_types=(
        [pltpu.VMEM((CHUNK * N_COLS,), jnp.float32) for _ in range(NBUF)]
        + [pltpu.SemaphoreType.DMA for _ in range(2 * NBUF)]
    ),
    compiler_params=pltpu.CompilerParams(
        use_tc_tiling_on_sc=False, needs_layout_passes=False
    ),
)
def _swap_columns(x_hbm, out_hbm, *scratch):
    bufs = scratch[:NBUF]
    in_sems = scratch[NBUF : 2 * NBUF]
    out_sems = scratch[2 * NBUF :]

    wid = lax.axis_index("s") * NUM_CORES + lax.axis_index("c")
    base = wid * (ROWS_PER_WORKER * N_COLS)

    def in_slice(i):
        return x_hbm.at[pl.ds(base + i * (CHUNK * N_COLS), CHUNK * N_COLS)]

    def out_slice(i):
        return out_hbm.at[pl.ds(base + i * (CHUNK * N_COLS), CHUNK * N_COLS)]

    in_h = [None] * NBUF
    out_h = [None] * NBUF
    for i in range(NUM_CHUNKS):
        b = i % NBUF
        if out_h[b] is not None:
            out_h[b].wait()
        in_h[b] = pltpu.async_copy(in_slice(i), bufs[b], in_sems[b])
        j = i - LAG
        if j >= 0:
            bj = j % NBUF
            in_h[bj].wait()
            _fix_columns(bufs[bj])
            out_h[bj] = pltpu.async_copy(bufs[bj], out_slice(j), out_sems[bj])
    for j in range(max(0, NUM_CHUNKS - LAG), NUM_CHUNKS):
        bj = j % NBUF
        in_h[bj].wait()
        _fix_columns(bufs[bj])
        out_h[bj] = pltpu.async_copy(bufs[bj], out_slice(j), out_sems[bj])
    for b in range(NBUF):
        if out_h[b] is not None:
            out_h[b].wait()


def kernel(X):
    return _swap_columns(X.reshape(-1)).reshape(N_ROWS, N_COLS)
